# Initial kernel scaffold; baseline (speedup 1.0000x reference)
import jax
import jax.numpy as jnp
from jax.experimental import pallas as pl


def kernel(hidden_states, top_k_index, top_k_weights, gate_up_proj, down_proj):
    raise NotImplementedError("write your pallas kernel here")


# speedup vs baseline: 3.2878x; 3.2878x over previous
import jax
import jax.numpy as jnp
from jax.experimental import pallas as pl
from jax.experimental.pallas import tpu as pltpu

_NUM_EXPERTS = 8
_QMAX = 127.0


def _fq_bf16(x):
    s = jnp.max(jnp.abs(x), axis=-1, keepdims=True) / _QMAX
    s = jnp.where(s <= 0.0, 1.0, s)
    return (jnp.round(x / s) * s).astype(jnp.bfloat16)


def _moe_dense_kernel(idx_ref, wt_ref, x_ref, gu_ref, dn_ref, out_ref):
    e = pl.program_id(1)

    q1 = _fq_bf16(x_ref[...])
    h = jax.lax.dot_general(
        q1, gu_ref[0], (((1,), (0,)), ((), ())),
        preferred_element_type=jnp.float32,
    )
    f = h.shape[-1] // 2
    gate = h[:, :f]
    up = h[:, f:]
    g = gate * jax.nn.sigmoid(gate) * up

    q2 = _fq_bf16(g)
    y = jax.lax.dot_general(
        q2, dn_ref[0], (((1,), (0,)), ((), ())),
        preferred_element_type=jnp.float32,
    )

    w = jnp.sum(
        jnp.where(idx_ref[...] == e, wt_ref[...], 0.0),
        axis=-1, keepdims=True,
    )

    @pl.when(e == 0)
    def _():
        out_ref[...] = w * y

    @pl.when(e > 0)
    def _():
        out_ref[...] += w * y


def kernel(hidden_states, top_k_index, top_k_weights, gate_up_proj, down_proj):
    n_tokens, d_model = hidden_states.shape
    n_experts, _, d_ff2 = gate_up_proj.shape
    top_k_index = top_k_index.astype(jnp.int32)

    gu_bf16 = gate_up_proj.astype(jnp.bfloat16)
    dn_bf16 = down_proj.astype(jnp.bfloat16)

    block_t = 1024
    grid = (n_tokens // block_t, n_experts)

    out = pl.pallas_call(
        _moe_dense_kernel,
        grid=grid,
        in_specs=[
            pl.BlockSpec((block_t, top_k_index.shape[1]), lambda t, e: (t, 0)),
            pl.BlockSpec((block_t, top_k_weights.shape[1]), lambda t, e: (t, 0)),
            pl.BlockSpec((block_t, d_model), lambda t, e: (t, 0)),
            pl.BlockSpec((1, d_model, d_ff2), lambda t, e: (e, 0, 0)),
            pl.BlockSpec((1, d_ff2 // 2, d_model), lambda t, e: (e, 0, 0)),
        ],
        out_specs=pl.BlockSpec((block_t, d_model), lambda t, e: (t, 0)),
        out_shape=jax.ShapeDtypeStruct((n_tokens, d_model), jnp.float32),
        compiler_params=pltpu.CompilerParams(
            dimension_semantics=("parallel", "arbitrary"),
        ),
    )(top_k_index, top_k_weights, hidden_states, gu_bf16, dn_bf16)
    return out
